# baseline (speedup 1.0000x reference)
import functools
import numpy as np
import jax
from jax import lax
import jax.numpy as jnp
from jax.experimental import pallas as pl
from jax.experimental.pallas import tpu as pltpu
from jax.experimental.pallas import tpu_sc as plsc

N = 62
F = 5
NH = 32
O1 = 64
NC = 3
NP = 64
NF = N * F
NFP = NP * F
NTRI = N * (N + 1) // 2
EWPAD = 2048
BN_EPS = 1e-5
_INV_SQRT1P = float(1.0 / np.sqrt(1.0 + BN_EPS))
_HI = jax.lax.Precision.HIGHEST

_ii = np.arange(NP)[:, None]
_jj = np.arange(NP)[None, :]
_hi_ = np.maximum(_ii, _jj)
_IDXMAP = np.where((_ii < N) & (_jj < N),
                   _hi_ * (_hi_ + 1) // 2 + np.minimum(_ii, _jj),
                   NTRI).astype(np.int32).reshape(-1)

_r = np.arange(NFP)
_c = np.arange(NFP)
_E_SEL = np.zeros((NFP, NP), np.float32)
_E_SEL[_r, _r // F] = 1.0
_MASK = ((_r[:, None] % F) == (_c[None, :] // O1)).astype(np.float32)
_JREP = ((_c[:, None] % O1) == np.arange(O1)[None, :]).astype(np.float32)

_INTERPRET = False


@functools.partial(
    pl.kernel,
    mesh=plsc.VectorSubcoreMesh(core_axis_name="c", subcore_axis_name="s"),
    out_type=jax.ShapeDtypeStruct((NP * NP,), jnp.float32),
    scratch_types=[pltpu.VMEM((EWPAD,), jnp.float32),
                   pltpu.VMEM((128,), jnp.int32),
                   pltpu.VMEM((128,), jnp.float32)],
    compiler_params=pltpu.CompilerParams(needs_layout_passes=False),
)
def _sc_scatter_build(ew_hbm, idx_hbm, out_hbm, ew_v, idx_v, row_v):
    wid = lax.axis_index("s") * 2 + lax.axis_index("c")
    pltpu.sync_copy(ew_hbm, ew_v)
    pltpu.sync_copy(idx_hbm.at[pl.ds(wid * 128, 128)], idx_v)
    for k in range(8):
        idx = idx_v[pl.ds(k * 16, 16)]
        vals = plsc.load_gather(ew_v, [idx])
        row_v[pl.ds(k * 16, 16)] = jnp.maximum(vals, 0.0)
    pltpu.sync_copy(row_v, out_hbm.at[pl.ds(wid * 128, 128)])


def _fold(a_ref, gmat_ref, e_ref, mask_ref, jrep_ref, gt_ref, bt_ref,
          bias1_ref, w_acc, c0_acc):
    a = a_ref[...]
    eye = (jax.lax.broadcasted_iota(jnp.int32, (NP, NP), 0) ==
           jax.lax.broadcasted_iota(jnp.int32, (NP, NP), 1)).astype(jnp.float32)
    drow = jnp.sum(a, axis=1, keepdims=True)
    dcol = jnp.sum(a, axis=0, keepdims=True)
    l = a * jax.lax.rsqrt(drow + 1e-10) * jax.lax.rsqrt(dcol + 1e-10)
    la = jnp.abs(l)
    deg_r = jnp.sum(la, axis=1, keepdims=True) + 1.0
    deg_c = jnp.sum(la, axis=0, keepdims=True) + 1.0
    s = (l + eye) * jax.lax.rsqrt(deg_r) * jax.lax.rsqrt(deg_c)
    p = jnp.dot(s, s, preferred_element_type=jnp.float32, precision=_HI)
    r = jnp.dot(p, gmat_ref[...], preferred_element_type=jnp.float32, precision=_HI)
    t1 = jnp.dot(e_ref[...], r, preferred_element_type=jnp.float32, precision=_HI)
    w0 = jnp.dot(t1 * mask_ref[...], jrep_ref[...],
                 preferred_element_type=jnp.float32, precision=_HI)
    w_acc[...] = w0 * (gt_ref[...] * _INV_SQRT1P)
    c0_acc[...] = (jnp.dot(bt_ref[...], w0, preferred_element_type=jnp.float32,
                           precision=_HI)
                   + bias1_ref[...])


def _fused_kernel(a_ref, gmat_ref, e_ref, mask_ref, jrep_ref, gt_ref, bt_ref,
                  bias1_ref, x_ref, w2_ref, b2_ref, o_ref, w_acc, c0_acc):
    @pl.when(pl.program_id(0) == 0)
    def _():
        _fold(a_ref, gmat_ref, e_ref, mask_ref, jrep_ref, gt_ref, bt_ref,
              bias1_ref, w_acc, c0_acc)

    w = w_acc[...]
    y = jnp.dot(x_ref[...], w[:NF, :], preferred_element_type=jnp.float32)
    y = jnp.maximum(y + c0_acc[...], 0.0)
    o_ref[...] = (jnp.dot(y, w2_ref[...], preferred_element_type=jnp.float32)
                  + b2_ref[...])


def kernel(X, edge_weight, bn_gamma, bn_beta, Wc, bc, W1, b1, W2, b2):
    B = X.shape[0]
    X_flat = X.reshape(B, NF)
    ew_pad = jnp.pad(edge_weight, (0, EWPAD - NTRI))
    a_dense = _sc_scatter_build(ew_pad, jnp.asarray(_IDXMAP)).reshape(NP, NP)
    W1r = W1.reshape(O1, N, NH)
    G = jnp.einsum('onh,hf->nfo', W1r, Wc)
    Gmat = jnp.pad(G.reshape(N, F * O1), ((0, NP - N), (0, 0)))
    bias1 = (b1 + jnp.einsum('onh,h->o', W1r, bc)).reshape(1, O1)
    gt = jnp.pad(jnp.tile(bn_gamma, N), (0, NFP - NF)).reshape(NFP, 1)
    bt = jnp.pad(jnp.tile(bn_beta, N), (0, NFP - NF)).reshape(1, NFP)

    BT = 256
    cblk = lambda i: (0, 0)
    out = pl.pallas_call(
        _fused_kernel,
        grid=(B // BT,),
        in_specs=[pl.BlockSpec((NP, NP), cblk),
                  pl.BlockSpec((NP, F * O1), cblk),
                  pl.BlockSpec((NFP, NP), cblk),
                  pl.BlockSpec((NFP, NFP), cblk),
                  pl.BlockSpec((NFP, O1), cblk),
                  pl.BlockSpec((NFP, 1), cblk),
                  pl.BlockSpec((1, NFP), cblk),
                  pl.BlockSpec((1, O1), cblk),
                  pl.BlockSpec((BT, NF), lambda i: (i, 0)),
                  pl.BlockSpec((O1, NC), cblk),
                  pl.BlockSpec((1, NC), cblk)],
        out_specs=pl.BlockSpec((BT, NC), lambda i: (i, 0)),
        out_shape=jax.ShapeDtypeStruct((B, NC), jnp.float32),
        scratch_shapes=[pltpu.VMEM((NFP, O1), jnp.float32),
                        pltpu.VMEM((1, O1), jnp.float32)],
        interpret=_INTERPRET,
    )(a_dense, Gmat, jnp.asarray(_E_SEL), jnp.asarray(_MASK), jnp.asarray(_JREP),
      gt, bt, bias1, X_flat, W2.T, b2.reshape(1, NC))
    return out


# speedup vs baseline: 1.0329x; 1.0329x over previous
import functools
import numpy as np
import jax
from jax import lax
import jax.numpy as jnp
from jax.experimental import pallas as pl
from jax.experimental.pallas import tpu as pltpu
from jax.experimental.pallas import tpu_sc as plsc

N = 62
F = 5
NH = 32
O1 = 64
NC = 3
NP = 64
NF = N * F
NFP = NP * F
NTRI = N * (N + 1) // 2
EWPAD = 2048
BN_EPS = 1e-5
_INV_SQRT1P = float(1.0 / np.sqrt(1.0 + BN_EPS))
_HI = jax.lax.Precision.HIGHEST

_ii = np.arange(NP)[:, None]
_jj = np.arange(NP)[None, :]
_hi_ = np.maximum(_ii, _jj)
_IDXMAP = np.where((_ii < N) & (_jj < N),
                   _hi_ * (_hi_ + 1) // 2 + np.minimum(_ii, _jj),
                   NTRI).astype(np.int32).reshape(-1)

_r = np.arange(NFP)
_c = np.arange(NFP)
_E_SEL = np.zeros((NFP, NP), np.float32)
_E_SEL[_r, _r // F] = 1.0
_MASK = ((_r[:, None] % F) == (_c[None, :] // O1)).astype(np.float32)
_JREP = ((_c[:, None] % O1) == np.arange(O1)[None, :]).astype(np.float32)

_INTERPRET = False


@functools.partial(
    pl.kernel,
    mesh=plsc.VectorSubcoreMesh(core_axis_name="c", subcore_axis_name="s"),
    out_type=jax.ShapeDtypeStruct((NP * NP,), jnp.float32),
    scratch_types=[pltpu.VMEM((EWPAD,), jnp.float32),
                   pltpu.VMEM((128,), jnp.int32),
                   pltpu.VMEM((128,), jnp.float32)],
    compiler_params=pltpu.CompilerParams(needs_layout_passes=False),
)
def _sc_scatter_build(ew_hbm, idx_hbm, out_hbm, ew_v, idx_v, row_v):
    wid = lax.axis_index("s") * 2 + lax.axis_index("c")
    pltpu.sync_copy(ew_hbm, ew_v)
    pltpu.sync_copy(idx_hbm.at[pl.ds(wid * 128, 128)], idx_v)
    for k in range(8):
        idx = idx_v[pl.ds(k * 16, 16)]
        vals = plsc.load_gather(ew_v, [idx])
        row_v[pl.ds(k * 16, 16)] = jnp.maximum(vals, 0.0)
    pltpu.sync_copy(row_v, out_hbm.at[pl.ds(wid * 128, 128)])


def _fold(a_ref, gmat_ref, e_ref, mask_ref, jrep_ref, gt_ref, bt_ref,
          bias1_ref, w_acc, c0_acc):
    a = a_ref[...]
    eye = (jax.lax.broadcasted_iota(jnp.int32, (NP, NP), 0) ==
           jax.lax.broadcasted_iota(jnp.int32, (NP, NP), 1)).astype(jnp.float32)
    drow = jnp.sum(a, axis=1, keepdims=True)
    dcol = jnp.sum(a, axis=0, keepdims=True)
    l = a * jax.lax.rsqrt(drow + 1e-10) * jax.lax.rsqrt(dcol + 1e-10)
    la = jnp.abs(l)
    deg_r = jnp.sum(la, axis=1, keepdims=True) + 1.0
    deg_c = jnp.sum(la, axis=0, keepdims=True) + 1.0
    s = (l + eye) * jax.lax.rsqrt(deg_r) * jax.lax.rsqrt(deg_c)
    p = jnp.dot(s, s, preferred_element_type=jnp.float32, precision=_HI)
    r = jnp.dot(p, gmat_ref[...], preferred_element_type=jnp.float32, precision=_HI)
    t1 = jnp.dot(e_ref[...], r, preferred_element_type=jnp.float32, precision=_HI)
    w0 = jnp.dot(t1 * mask_ref[...], jrep_ref[...],
                 preferred_element_type=jnp.float32, precision=_HI)

    w_acc[...] = w0 * (gt_ref[...] * _INV_SQRT1P)
    c0_acc[...] = (jnp.dot(bt_ref[...], w0, preferred_element_type=jnp.float32,
                           precision=_HI)
                   + bias1_ref[...])


def _fused_kernel(a_ref, gmat_ref, e_ref, mask_ref, jrep_ref, gt_ref, bt_ref,
                  bias1_ref, x_ref, w2_ref, b2_ref, o_ref, w_acc, c0_acc):
    @pl.when(pl.program_id(0) == 0)
    def _():
        _fold(a_ref, gmat_ref, e_ref, mask_ref, jrep_ref, gt_ref, bt_ref,
              bias1_ref, w_acc, c0_acc)

    w = w_acc[...]
    y = jnp.dot(x_ref[...], w[:NF, :], preferred_element_type=jnp.float32)
    y = jnp.maximum(y + c0_acc[...], 0.0)
    o_ref[...] = (jnp.dot(y, w2_ref[...], preferred_element_type=jnp.float32)
                  + b2_ref[...])


def kernel(X, edge_weight, bn_gamma, bn_beta, Wc, bc, W1, b1, W2, b2):
    B = X.shape[0]
    X_flat = X.reshape(B, NF)
    ew_pad = jnp.pad(edge_weight, (0, EWPAD - NTRI))
    a_dense = _sc_scatter_build(ew_pad, jnp.asarray(_IDXMAP)).reshape(NP, NP)
    W1r = W1.reshape(O1, N, NH)
    G = jnp.einsum('onh,hf->nfo', W1r, Wc)
    Gmat = jnp.pad(G.reshape(N, F * O1), ((0, NP - N), (0, 0)))
    bias1 = (b1 + jnp.einsum('onh,h->o', W1r, bc)).reshape(1, O1)
    gt = jnp.pad(jnp.tile(bn_gamma, N), (0, NFP - NF)).reshape(NFP, 1)
    bt = jnp.pad(jnp.tile(bn_beta, N), (0, NFP - NF)).reshape(1, NFP)

    BT = 512
    cblk = lambda i: (0, 0)
    out = pl.pallas_call(
        _fused_kernel,
        grid=(B // BT,),
        in_specs=[pl.BlockSpec((NP, NP), cblk),
                  pl.BlockSpec((NP, F * O1), cblk),
                  pl.BlockSpec((NFP, NP), cblk),
                  pl.BlockSpec((NFP, NFP), cblk),
                  pl.BlockSpec((NFP, O1), cblk),
                  pl.BlockSpec((NFP, 1), cblk),
                  pl.BlockSpec((1, NFP), cblk),
                  pl.BlockSpec((1, O1), cblk),
                  pl.BlockSpec((BT, NF), lambda i: (i, 0)),
                  pl.BlockSpec((O1, NC), cblk),
                  pl.BlockSpec((1, NC), cblk)],
        out_specs=pl.BlockSpec((BT, NC), lambda i: (i, 0)),
        out_shape=jax.ShapeDtypeStruct((B, NC), jnp.float32),
        scratch_shapes=[pltpu.VMEM((NFP, O1), jnp.float32),
                        pltpu.VMEM((1, O1), jnp.float32)],
        interpret=_INTERPRET,
    )(a_dense, Gmat, jnp.asarray(_E_SEL), jnp.asarray(_MASK), jnp.asarray(_JREP),
      gt, bt, bias1, X_flat, W2.T, b2.reshape(1, NC))
    return out


# speedup vs baseline: 1.0417x; 1.0085x over previous
import functools
import numpy as np
import jax
from jax import lax
import jax.numpy as jnp
from jax.experimental import pallas as pl
from jax.experimental.pallas import tpu as pltpu
from jax.experimental.pallas import tpu_sc as plsc

N = 62
F = 5
NH = 32
O1 = 64
NC = 3
NP = 64
NF = N * F
NFP = NP * F
NTRI = N * (N + 1) // 2
EWPAD = 2048
BN_EPS = 1e-5
_INV_SQRT1P = float(1.0 / np.sqrt(1.0 + BN_EPS))
_HI = jax.lax.Precision.HIGHEST

_ii = np.arange(NP)[:, None]
_jj = np.arange(NP)[None, :]
_hi_ = np.maximum(_ii, _jj)
_IDXMAP = np.where((_ii < N) & (_jj < N),
                   _hi_ * (_hi_ + 1) // 2 + np.minimum(_ii, _jj),
                   NTRI).astype(np.int32).reshape(-1)

_r = np.arange(NFP)
_c = np.arange(NFP)
_E_SEL = np.zeros((NFP, NP), np.float32)
_E_SEL[_r, _r // F] = 1.0
_MASK = ((_r[:, None] % F) == (_c[None, :] // O1)).astype(np.float32)
_JREP = ((_c[:, None] % O1) == np.arange(O1)[None, :]).astype(np.float32)

_INTERPRET = False


@functools.partial(
    pl.kernel,
    mesh=plsc.VectorSubcoreMesh(core_axis_name="c", subcore_axis_name="s"),
    out_type=jax.ShapeDtypeStruct((NP * NP,), jnp.float32),
    scratch_types=[pltpu.VMEM((EWPAD,), jnp.float32),
                   pltpu.VMEM((128,), jnp.int32),
                   pltpu.VMEM((128,), jnp.float32)],
    compiler_params=pltpu.CompilerParams(needs_layout_passes=False),
)
def _sc_scatter_build(ew_hbm, idx_hbm, out_hbm, ew_v, idx_v, row_v):
    wid = lax.axis_index("s") * 2 + lax.axis_index("c")
    pltpu.sync_copy(ew_hbm, ew_v)
    pltpu.sync_copy(idx_hbm.at[pl.ds(wid * 128, 128)], idx_v)
    for k in range(8):
        idx = idx_v[pl.ds(k * 16, 16)]
        vals = plsc.load_gather(ew_v, [idx])
        row_v[pl.ds(k * 16, 16)] = jnp.maximum(vals, 0.0)
    pltpu.sync_copy(row_v, out_hbm.at[pl.ds(wid * 128, 128)])


def _fold(a_ref, gmat_ref, e_ref, mask_ref, jrep_ref, gt_ref, bt_ref,
          bias1_ref, w_acc, c0_acc):
    a = a_ref[...]
    eye = (jax.lax.broadcasted_iota(jnp.int32, (NP, NP), 0) ==
           jax.lax.broadcasted_iota(jnp.int32, (NP, NP), 1)).astype(jnp.float32)
    drow = jnp.sum(a, axis=1, keepdims=True)
    dcol = jnp.sum(a, axis=0, keepdims=True)
    l = a * jax.lax.rsqrt(drow + 1e-10) * jax.lax.rsqrt(dcol + 1e-10)
    la = jnp.abs(l)
    deg_r = jnp.sum(la, axis=1, keepdims=True) + 1.0
    deg_c = jnp.sum(la, axis=0, keepdims=True) + 1.0
    s = (l + eye) * jax.lax.rsqrt(deg_r) * jax.lax.rsqrt(deg_c)
    p = jnp.dot(s, s, preferred_element_type=jnp.float32, precision=_HI)
    r = jnp.dot(p, gmat_ref[...], preferred_element_type=jnp.float32, precision=_HI)
    t1 = jnp.dot(e_ref[...], r, preferred_element_type=jnp.float32, precision=_HI)
    w0 = jnp.dot(t1 * mask_ref[...], jrep_ref[...],
                 preferred_element_type=jnp.float32, precision=_HI)

    w_acc[...] = w0 * (gt_ref[...] * _INV_SQRT1P)
    c0_acc[...] = (jnp.dot(bt_ref[...], w0, preferred_element_type=jnp.float32,
                           precision=_HI)
                   + bias1_ref[...])


def _fused_kernel(a_ref, gmat_ref, e_ref, mask_ref, jrep_ref, gt_ref, bt_ref,
                  bias1_ref, x_ref, w2_ref, b2_ref, o_ref, w_acc, c0_acc):
    @pl.when(pl.program_id(0) == 0)
    def _():
        _fold(a_ref, gmat_ref, e_ref, mask_ref, jrep_ref, gt_ref, bt_ref,
              bias1_ref, w_acc, c0_acc)

    w = w_acc[...]
    y = jnp.dot(x_ref[...], w[:NF, :], preferred_element_type=jnp.float32)
    y = jnp.maximum(y + c0_acc[...], 0.0)
    o_ref[...] = (jnp.dot(y, w2_ref[...], preferred_element_type=jnp.float32)
                  + b2_ref[...])


def kernel(X, edge_weight, bn_gamma, bn_beta, Wc, bc, W1, b1, W2, b2):
    B = X.shape[0]
    X_flat = X.reshape(B, NF)
    ew_pad = jnp.pad(edge_weight, (0, EWPAD - NTRI))
    a_dense = _sc_scatter_build(ew_pad, jnp.asarray(_IDXMAP)).reshape(NP, NP)
    W1r = W1.reshape(O1, N, NH)
    G = jnp.einsum('onh,hf->nfo', W1r, Wc)
    Gmat = jnp.pad(G.reshape(N, F * O1), ((0, NP - N), (0, 0)))
    bias1 = (b1 + jnp.einsum('onh,h->o', W1r, bc)).reshape(1, O1)
    gt = jnp.pad(jnp.tile(bn_gamma, N), (0, NFP - NF)).reshape(NFP, 1)
    bt = jnp.pad(jnp.tile(bn_beta, N), (0, NFP - NF)).reshape(1, NFP)

    BT = 1024
    cblk = lambda i: (0, 0)
    out = pl.pallas_call(
        _fused_kernel,
        grid=(B // BT,),
        in_specs=[pl.BlockSpec((NP, NP), cblk),
                  pl.BlockSpec((NP, F * O1), cblk),
                  pl.BlockSpec((NFP, NP), cblk),
                  pl.BlockSpec((NFP, NFP), cblk),
                  pl.BlockSpec((NFP, O1), cblk),
                  pl.BlockSpec((NFP, 1), cblk),
                  pl.BlockSpec((1, NFP), cblk),
                  pl.BlockSpec((1, O1), cblk),
                  pl.BlockSpec((BT, NF), lambda i: (i, 0)),
                  pl.BlockSpec((O1, NC), cblk),
                  pl.BlockSpec((1, NC), cblk)],
        out_specs=pl.BlockSpec((BT, NC), lambda i: (i, 0)),
        out_shape=jax.ShapeDtypeStruct((B, NC), jnp.float32),
        scratch_shapes=[pltpu.VMEM((NFP, O1), jnp.float32),
                        pltpu.VMEM((1, O1), jnp.float32)],
        interpret=_INTERPRET,
    )(a_dense, Gmat, jnp.asarray(_E_SEL), jnp.asarray(_MASK), jnp.asarray(_JREP),
      gt, bt, bias1, X_flat, W2.T, b2.reshape(1, NC))
    return out


# speedup vs baseline: 1.1144x; 1.0698x over previous
import functools
import numpy as np
import jax
from jax import lax
import jax.numpy as jnp
from jax.experimental import pallas as pl
from jax.experimental.pallas import tpu as pltpu
from jax.experimental.pallas import tpu_sc as plsc

N = 62
F = 5
NH = 32
O1 = 64
NC = 3
NP = 64
NF = N * F
NFP = NP * F
NTRI = N * (N + 1) // 2
EWPAD = 2048
BN_EPS = 1e-5
_INV_SQRT1P = float(1.0 / np.sqrt(1.0 + BN_EPS))
_HI = jax.lax.Precision.HIGHEST

_ii = np.arange(NP)[:, None]
_jj = np.arange(NP)[None, :]
_hi_ = np.maximum(_ii, _jj)
_IDXMAP = np.where((_ii < N) & (_jj < N),
                   _hi_ * (_hi_ + 1) // 2 + np.minimum(_ii, _jj),
                   NTRI).astype(np.int32).reshape(-1)

_r = np.arange(NFP)
_c = np.arange(NFP)
_E_SEL = np.zeros((NFP, NP), np.float32)
_E_SEL[_r, _r // F] = 1.0
_MASK = ((_r[:, None] % F) == (_c[None, :] // O1)).astype(np.float32)
_JREP = ((_c[:, None] % O1) == np.arange(O1)[None, :]).astype(np.float32)

_INTERPRET = False


@functools.partial(
    pl.kernel,
    mesh=plsc.VectorSubcoreMesh(core_axis_name="c", subcore_axis_name="s"),
    out_type=jax.ShapeDtypeStruct((NP * NP,), jnp.float32),
    scratch_types=[pltpu.VMEM((EWPAD,), jnp.float32),
                   pltpu.VMEM((128,), jnp.float32)],
    compiler_params=pltpu.CompilerParams(needs_layout_passes=False),
)
def _sc_scatter_build(ew_hbm, out_hbm, ew_v, row_v):
    wid = lax.axis_index("s") * 2 + lax.axis_index("c")
    pltpu.sync_copy(ew_hbm, ew_v)
    for rr in range(2):
        i = wid * 2 + rr
        tri_i = (i * (i + 1)) // 2
        for k in range(4):
            j = lax.iota(jnp.int32, 16) + (k * 16)
            low = tri_i + j
            high = ((j * (j + 1)) // 2) + i
            idx = jnp.where(j <= i, low, high)
            idx = jnp.where((j < N) & (i < N), idx, NTRI)
            vals = plsc.load_gather(ew_v, [idx])
            row_v[pl.ds(rr * 64 + k * 16, 16)] = jnp.maximum(vals, 0.0)
    pltpu.sync_copy(row_v, out_hbm.at[pl.ds(wid * 128, 128)])


def _fold(a_ref, gmat_ref, e_ref, mask_ref, jrep_ref, gt_ref, bt_ref,
          bias1_ref, w_acc, c0_acc):
    a = a_ref[...]
    eye = (jax.lax.broadcasted_iota(jnp.int32, (NP, NP), 0) ==
           jax.lax.broadcasted_iota(jnp.int32, (NP, NP), 1)).astype(jnp.float32)
    drow = jnp.sum(a, axis=1, keepdims=True)
    dcol = jnp.sum(a, axis=0, keepdims=True)
    l = a * jax.lax.rsqrt(drow + 1e-10) * jax.lax.rsqrt(dcol + 1e-10)
    la = jnp.abs(l)
    deg_r = jnp.sum(la, axis=1, keepdims=True) + 1.0
    deg_c = jnp.sum(la, axis=0, keepdims=True) + 1.0
    s = (l + eye) * jax.lax.rsqrt(deg_r) * jax.lax.rsqrt(deg_c)
    p = jnp.dot(s, s, preferred_element_type=jnp.float32, precision=_HI)
    r = jnp.dot(p, gmat_ref[...], preferred_element_type=jnp.float32, precision=_HI)
    t1 = jnp.dot(e_ref[...], r, preferred_element_type=jnp.float32, precision=_HI)
    w0 = jnp.dot(t1 * mask_ref[...], jrep_ref[...],
                 preferred_element_type=jnp.float32, precision=_HI)

    w_acc[...] = w0 * (gt_ref[...] * _INV_SQRT1P)
    c0_acc[...] = (jnp.dot(bt_ref[...], w0, preferred_element_type=jnp.float32,
                           precision=_HI)
                   + bias1_ref[...])


def _fused_kernel(a_ref, gmat_ref, e_ref, mask_ref, jrep_ref, gt_ref, bt_ref,
                  bias1_ref, x_ref, w2_ref, b2_ref, o_ref, w_acc, c0_acc):
    @pl.when(pl.program_id(0) == 0)
    def _():
        _fold(a_ref, gmat_ref, e_ref, mask_ref, jrep_ref, gt_ref, bt_ref,
              bias1_ref, w_acc, c0_acc)

    w = w_acc[...]
    y = jnp.dot(x_ref[...], w[:NF, :], preferred_element_type=jnp.float32)
    y = jnp.maximum(y + c0_acc[...], 0.0)
    o_ref[...] = (jnp.dot(y, w2_ref[...], preferred_element_type=jnp.float32)
                  + b2_ref[...])


def kernel(X, edge_weight, bn_gamma, bn_beta, Wc, bc, W1, b1, W2, b2):
    B = X.shape[0]
    X_flat = X.reshape(B, NF)
    ew_pad = jnp.pad(edge_weight, (0, EWPAD - NTRI))
    a_dense = _sc_scatter_build(ew_pad).reshape(NP, NP)
    W1r = W1.reshape(O1, N, NH)
    G = jnp.einsum('onh,hf->nfo', W1r, Wc)
    Gmat = jnp.pad(G.reshape(N, F * O1), ((0, NP - N), (0, 0)))
    bias1 = (b1 + jnp.einsum('onh,h->o', W1r, bc)).reshape(1, O1)
    gt = jnp.pad(jnp.tile(bn_gamma, N), (0, NFP - NF)).reshape(NFP, 1)
    bt = jnp.pad(jnp.tile(bn_beta, N), (0, NFP - NF)).reshape(1, NFP)

    BT = 1024
    cblk = lambda i: (0, 0)
    out = pl.pallas_call(
        _fused_kernel,
        grid=(B // BT,),
        in_specs=[pl.BlockSpec((NP, NP), cblk),
                  pl.BlockSpec((NP, F * O1), cblk),
                  pl.BlockSpec((NFP, NP), cblk),
                  pl.BlockSpec((NFP, NFP), cblk),
                  pl.BlockSpec((NFP, O1), cblk),
                  pl.BlockSpec((NFP, 1), cblk),
                  pl.BlockSpec((1, NFP), cblk),
                  pl.BlockSpec((1, O1), cblk),
                  pl.BlockSpec((BT, NF), lambda i: (i, 0)),
                  pl.BlockSpec((O1, NC), cblk),
                  pl.BlockSpec((1, NC), cblk)],
        out_specs=pl.BlockSpec((BT, NC), lambda i: (i, 0)),
        out_shape=jax.ShapeDtypeStruct((B, NC), jnp.float32),
        scratch_shapes=[pltpu.VMEM((NFP, O1), jnp.float32),
                        pltpu.VMEM((1, O1), jnp.float32)],
        interpret=_INTERPRET,
    )(a_dense, Gmat, jnp.asarray(_E_SEL), jnp.asarray(_MASK), jnp.asarray(_JREP),
      gt, bt, bias1, X_flat, W2.T, b2.reshape(1, NC))
    return out


# speedup vs baseline: 1.1145x; 1.0001x over previous
import functools
import numpy as np
import jax
from jax import lax
import jax.numpy as jnp
from jax.experimental import pallas as pl
from jax.experimental.pallas import tpu as pltpu
from jax.experimental.pallas import tpu_sc as plsc

N = 62
F = 5
NH = 32
O1 = 64
NC = 3
NP = 64
NF = N * F
NFP = NP * F
NTRI = N * (N + 1) // 2
EWPAD = 2048
BN_EPS = 1e-5
_INV_SQRT1P = float(1.0 / np.sqrt(1.0 + BN_EPS))
_HI = jax.lax.Precision.HIGHEST

_r = np.arange(NFP)
_c = np.arange(NFP)
_E_SEL = np.zeros((NFP, NP), np.float32)
_E_SEL[_r, _r // F] = 1.0
_MASK = ((_r[:, None] % F) == (_c[None, :] // O1)).astype(np.float32)
_JREP = ((_c[:, None] % O1) == np.arange(O1)[None, :]).astype(np.float32)

@functools.partial(
    pl.kernel,
    mesh=plsc.VectorSubcoreMesh(core_axis_name="c", subcore_axis_name="s"),
    out_type=jax.ShapeDtypeStruct((NP * NP,), jnp.float32),
    scratch_types=[pltpu.VMEM((EWPAD,), jnp.float32),
                   pltpu.VMEM((128,), jnp.float32)],
    compiler_params=pltpu.CompilerParams(needs_layout_passes=False),
)
def _sc_scatter_build(ew_hbm, out_hbm, ew_v, row_v):
    wid = lax.axis_index("s") * 2 + lax.axis_index("c")
    pltpu.sync_copy(ew_hbm, ew_v)
    for rr in range(2):
        i = wid * 2 + rr
        tri_i = (i * (i + 1)) // 2
        for k in range(4):
            j = lax.iota(jnp.int32, 16) + (k * 16)
            low = tri_i + j
            high = ((j * (j + 1)) // 2) + i
            idx = jnp.where(j <= i, low, high)
            idx = jnp.where((j < N) & (i < N), idx, NTRI)
            vals = plsc.load_gather(ew_v, [idx])
            row_v[pl.ds(rr * 64 + k * 16, 16)] = jnp.maximum(vals, 0.0)
    pltpu.sync_copy(row_v, out_hbm.at[pl.ds(wid * 128, 128)])


def _fold(a_ref, gmat_ref, e_ref, mask_ref, jrep_ref, gt_ref, bt_ref,
          bias1_ref, w_acc, c0_acc):
    a = a_ref[...]
    eye = (jax.lax.broadcasted_iota(jnp.int32, (NP, NP), 0) ==
           jax.lax.broadcasted_iota(jnp.int32, (NP, NP), 1)).astype(jnp.float32)
    drow = jnp.sum(a, axis=1, keepdims=True)
    dcol = jnp.sum(a, axis=0, keepdims=True)
    l = a * jax.lax.rsqrt(drow + 1e-10) * jax.lax.rsqrt(dcol + 1e-10)
    la = jnp.abs(l)
    deg_r = jnp.sum(la, axis=1, keepdims=True) + 1.0
    deg_c = jnp.sum(la, axis=0, keepdims=True) + 1.0
    s = (l + eye) * jax.lax.rsqrt(deg_r) * jax.lax.rsqrt(deg_c)
    p = jnp.dot(s, s, preferred_element_type=jnp.float32, precision=_HI)
    r = jnp.dot(p, gmat_ref[...], preferred_element_type=jnp.float32, precision=_HI)
    t1 = jnp.dot(e_ref[...], r, preferred_element_type=jnp.float32, precision=_HI)
    w0 = jnp.dot(t1 * mask_ref[...], jrep_ref[...],
                 preferred_element_type=jnp.float32, precision=_HI)

    w_acc[...] = w0 * (gt_ref[...] * _INV_SQRT1P)
    c0_acc[...] = (jnp.dot(bt_ref[...], w0, preferred_element_type=jnp.float32,
                           precision=_HI)
                   + bias1_ref[...])


def _fused_kernel(a_ref, gmat_ref, e_ref, mask_ref, jrep_ref, gt_ref, bt_ref,
                  bias1_ref, x_ref, w2_ref, b2_ref, o_ref, w_acc, c0_acc):
    @pl.when(pl.program_id(0) == 0)
    def _():
        _fold(a_ref, gmat_ref, e_ref, mask_ref, jrep_ref, gt_ref, bt_ref,
              bias1_ref, w_acc, c0_acc)

    w = w_acc[...]
    y = jnp.dot(x_ref[...], w[:NF, :], preferred_element_type=jnp.float32)
    y = jnp.maximum(y + c0_acc[...], 0.0)
    o_ref[...] = (jnp.dot(y, w2_ref[...], preferred_element_type=jnp.float32)
                  + b2_ref[...])


def kernel(X, edge_weight, bn_gamma, bn_beta, Wc, bc, W1, b1, W2, b2):
    B = X.shape[0]
    X_flat = X.reshape(B, NF)
    ew_pad = jnp.pad(edge_weight, (0, EWPAD - NTRI))
    a_dense = _sc_scatter_build(ew_pad).reshape(NP, NP)
    W1r = W1.reshape(O1, N, NH)
    G = jnp.einsum('onh,hf->nfo', W1r, Wc)
    Gmat = jnp.pad(G.reshape(N, F * O1), ((0, NP - N), (0, 0)))
    bias1 = (b1 + jnp.einsum('onh,h->o', W1r, bc)).reshape(1, O1)
    gt = jnp.pad(jnp.tile(bn_gamma, N), (0, NFP - NF)).reshape(NFP, 1)
    bt = jnp.pad(jnp.tile(bn_beta, N), (0, NFP - NF)).reshape(1, NFP)

    BT = 1024
    cblk = lambda i: (0, 0)
    out = pl.pallas_call(
        _fused_kernel,
        grid=(B // BT,),
        in_specs=[pl.BlockSpec((NP, NP), cblk),
                  pl.BlockSpec((NP, F * O1), cblk),
                  pl.BlockSpec((NFP, NP), cblk),
                  pl.BlockSpec((NFP, NFP), cblk),
                  pl.BlockSpec((NFP, O1), cblk),
                  pl.BlockSpec((NFP, 1), cblk),
                  pl.BlockSpec((1, NFP), cblk),
                  pl.BlockSpec((1, O1), cblk),
                  pl.BlockSpec((BT, NF), lambda i: (i, 0)),
                  pl.BlockSpec((O1, NC), cblk),
                  pl.BlockSpec((1, NC), cblk)],
        out_specs=pl.BlockSpec((BT, NC), lambda i: (i, 0)),
        out_shape=jax.ShapeDtypeStruct((B, NC), jnp.float32),
        scratch_shapes=[pltpu.VMEM((NFP, O1), jnp.float32),
                        pltpu.VMEM((1, O1), jnp.float32)],
    )(a_dense, Gmat, jnp.asarray(_E_SEL), jnp.asarray(_MASK), jnp.asarray(_JREP),
      gt, bt, bias1, X_flat, W2.T, b2.reshape(1, NC))
    return out


# speedup vs baseline: 1.1659x; 1.0460x over previous
import functools
import numpy as np
import jax
from jax import lax
import jax.numpy as jnp
from jax.experimental import pallas as pl
from jax.experimental.pallas import tpu as pltpu
from jax.experimental.pallas import tpu_sc as plsc

N = 62
F = 5
NH = 32
O1 = 64
NC = 3
NP = 64
NF = N * F
NFP = NP * F
NTRI = N * (N + 1) // 2
EWPAD = 2048
BN_EPS = 1e-5
_INV_SQRT1P = float(1.0 / np.sqrt(1.0 + BN_EPS))
_HI = jax.lax.Precision.HIGHEST

_r = np.arange(NFP)
_c = np.arange(NFP)
_E_SEL = np.zeros((NFP, NP), np.float32)
_E_SEL[_r, _r // F] = 1.0
_MASK = ((_r[:, None] % F) == (_c[None, :] // O1)).astype(np.float32)
_JREP = ((_c[:, None] % O1) == np.arange(O1)[None, :]).astype(np.float32)

@functools.partial(
    pl.kernel,
    mesh=plsc.VectorSubcoreMesh(core_axis_name="c", subcore_axis_name="s", num_cores=1),
    out_type=jax.ShapeDtypeStruct((NP * NP,), jnp.float32),
    scratch_types=[pltpu.VMEM((EWPAD,), jnp.float32),
                   pltpu.VMEM((256,), jnp.float32)],
    compiler_params=pltpu.CompilerParams(needs_layout_passes=False),
)
def _sc_scatter_build(ew_hbm, out_hbm, ew_v, row_v):
    wid = lax.axis_index("s")
    pltpu.sync_copy(ew_hbm, ew_v)
    for rr in range(4):
        i = wid * 4 + rr
        tri_i = (i * (i + 1)) // 2
        for k in range(4):
            j = lax.iota(jnp.int32, 16) + (k * 16)
            low = tri_i + j
            high = ((j * (j + 1)) // 2) + i
            idx = jnp.where(j <= i, low, high)
            idx = jnp.where((j < N) & (i < N), idx, NTRI)
            vals = plsc.load_gather(ew_v, [idx])
            row_v[pl.ds(rr * 64 + k * 16, 16)] = jnp.maximum(vals, 0.0)
    pltpu.sync_copy(row_v, out_hbm.at[pl.ds(wid * 256, 256)])


def _fold(a_ref, gmat_ref, e_ref, mask_ref, jrep_ref, gt_ref, bt_ref,
          bias1_ref, w_acc, c0_acc):
    a = a_ref[...]
    eye = (jax.lax.broadcasted_iota(jnp.int32, (NP, NP), 0) ==
           jax.lax.broadcasted_iota(jnp.int32, (NP, NP), 1)).astype(jnp.float32)
    drow = jnp.sum(a, axis=1, keepdims=True)
    dcol = jnp.sum(a, axis=0, keepdims=True)
    l = a * jax.lax.rsqrt(drow + 1e-10) * jax.lax.rsqrt(dcol + 1e-10)
    la = jnp.abs(l)
    deg_r = jnp.sum(la, axis=1, keepdims=True) + 1.0
    deg_c = jnp.sum(la, axis=0, keepdims=True) + 1.0
    s = (l + eye) * jax.lax.rsqrt(deg_r) * jax.lax.rsqrt(deg_c)
    p = jnp.dot(s, s, preferred_element_type=jnp.float32, precision=_HI)
    r = jnp.dot(p, gmat_ref[...], preferred_element_type=jnp.float32, precision=_HI)
    t1 = jnp.dot(e_ref[...], r, preferred_element_type=jnp.float32, precision=_HI)
    w0 = jnp.dot(t1 * mask_ref[...], jrep_ref[...],
                 preferred_element_type=jnp.float32, precision=_HI)

    w_acc[...] = w0 * (gt_ref[...] * _INV_SQRT1P)
    c0_acc[...] = (jnp.dot(bt_ref[...], w0, preferred_element_type=jnp.float32,
                           precision=_HI)
                   + bias1_ref[...])


def _fused_kernel(a_ref, gmat_ref, e_ref, mask_ref, jrep_ref, gt_ref, bt_ref,
                  bias1_ref, x_ref, w2_ref, b2_ref, o_ref, w_acc, c0_acc):
    @pl.when(pl.program_id(0) == 0)
    def _():
        _fold(a_ref, gmat_ref, e_ref, mask_ref, jrep_ref, gt_ref, bt_ref,
              bias1_ref, w_acc, c0_acc)

    w = w_acc[...]
    y = jnp.dot(x_ref[...], w[:NF, :], preferred_element_type=jnp.float32)
    y = jnp.maximum(y + c0_acc[...], 0.0)
    o_ref[...] = (jnp.dot(y, w2_ref[...], preferred_element_type=jnp.float32)
                  + b2_ref[...])


def kernel(X, edge_weight, bn_gamma, bn_beta, Wc, bc, W1, b1, W2, b2):
    B = X.shape[0]
    X_flat = X.reshape(B, NF)
    ew_pad = jnp.pad(edge_weight, (0, EWPAD - NTRI))
    a_dense = _sc_scatter_build(ew_pad).reshape(NP, NP)
    W1r = W1.reshape(O1, N, NH)
    G = jnp.einsum('onh,hf->nfo', W1r, Wc)
    Gmat = jnp.pad(G.reshape(N, F * O1), ((0, NP - N), (0, 0)))
    bias1 = (b1 + jnp.einsum('onh,h->o', W1r, bc)).reshape(1, O1)
    gt = jnp.pad(jnp.tile(bn_gamma, N), (0, NFP - NF)).reshape(NFP, 1)
    bt = jnp.pad(jnp.tile(bn_beta, N), (0, NFP - NF)).reshape(1, NFP)

    BT = 1024
    cblk = lambda i: (0, 0)
    out = pl.pallas_call(
        _fused_kernel,
        grid=(B // BT,),
        in_specs=[pl.BlockSpec((NP, NP), cblk),
                  pl.BlockSpec((NP, F * O1), cblk),
                  pl.BlockSpec((NFP, NP), cblk),
                  pl.BlockSpec((NFP, NFP), cblk),
                  pl.BlockSpec((NFP, O1), cblk),
                  pl.BlockSpec((NFP, 1), cblk),
                  pl.BlockSpec((1, NFP), cblk),
                  pl.BlockSpec((1, O1), cblk),
                  pl.BlockSpec((BT, NF), lambda i: (i, 0)),
                  pl.BlockSpec((O1, NC), cblk),
                  pl.BlockSpec((1, NC), cblk)],
        out_specs=pl.BlockSpec((BT, NC), lambda i: (i, 0)),
        out_shape=jax.ShapeDtypeStruct((B, NC), jnp.float32),
        scratch_shapes=[pltpu.VMEM((NFP, O1), jnp.float32),
                        pltpu.VMEM((1, O1), jnp.float32)],
    )(a_dense, Gmat, jnp.asarray(_E_SEL), jnp.asarray(_MASK), jnp.asarray(_JREP),
      gt, bt, bias1, X_flat, W2.T, b2.reshape(1, NC))
    return out


# speedup vs baseline: 1.1882x; 1.0191x over previous
import functools
import numpy as np
import jax
from jax import lax
import jax.numpy as jnp
from jax.experimental import pallas as pl
from jax.experimental.pallas import tpu as pltpu
from jax.experimental.pallas import tpu_sc as plsc

N = 62
F = 5
NH = 32
O1 = 64
NC = 3
NP = 64
NF = N * F
NFP = NP * F
NTRI = N * (N + 1) // 2
BN_EPS = 1e-5
_INV_SQRT1P = float(1.0 / np.sqrt(1.0 + BN_EPS))
_HI = jax.lax.Precision.HIGHEST

_r = np.arange(NFP)
_c = np.arange(NFP)
_E_SEL = np.zeros((NFP, NP), np.float32)
_E_SEL[_r, _r // F] = 1.0
_MASK = ((_r[:, None] % F) == (_c[None, :] // O1)).astype(np.float32)
_JREP = ((_c[:, None] % O1) == np.arange(O1)[None, :]).astype(np.float32)

@functools.partial(
    pl.kernel,
    mesh=plsc.VectorSubcoreMesh(core_axis_name="c", subcore_axis_name="s", num_cores=1),
    out_type=jax.ShapeDtypeStruct((NP * NP,), jnp.float32),
    scratch_types=[pltpu.VMEM((NTRI,), jnp.float32),
                   pltpu.VMEM((256,), jnp.float32)],
    compiler_params=pltpu.CompilerParams(needs_layout_passes=False),
)
def _sc_scatter_build(ew_hbm, out_hbm, ew_v, row_v):
    wid = lax.axis_index("s")
    pltpu.sync_copy(ew_hbm, ew_v)
    for rr in range(4):
        i = wid * 4 + rr
        tri_i = (i * (i + 1)) // 2
        for k in range(4):
            j = lax.iota(jnp.int32, 16) + (k * 16)
            low = tri_i + j
            high = ((j * (j + 1)) // 2) + i
            idx = jnp.where(j <= i, low, high)
            valid = (j < N) & (i < N)
            idx = jnp.where(valid, idx, 0)
            vals = plsc.load_gather(ew_v, [idx])
            vals = jnp.where(valid, jnp.maximum(vals, 0.0), 0.0)
            row_v[pl.ds(rr * 64 + k * 16, 16)] = vals
    pltpu.sync_copy(row_v, out_hbm.at[pl.ds(wid * 256, 256)])


def _fold(a_ref, gmat_ref, e_ref, mask_ref, jrep_ref, gt_ref, bt_ref,
          bias1_ref, w_acc, c0_acc):
    a = a_ref[...]
    eye = (jax.lax.broadcasted_iota(jnp.int32, (NP, NP), 0) ==
           jax.lax.broadcasted_iota(jnp.int32, (NP, NP), 1)).astype(jnp.float32)
    drow = jnp.sum(a, axis=1, keepdims=True)
    dcol = jnp.sum(a, axis=0, keepdims=True)
    l = a * jax.lax.rsqrt(drow + 1e-10) * jax.lax.rsqrt(dcol + 1e-10)
    la = jnp.abs(l)
    deg_r = jnp.sum(la, axis=1, keepdims=True) + 1.0
    deg_c = jnp.sum(la, axis=0, keepdims=True) + 1.0
    s = (l + eye) * jax.lax.rsqrt(deg_r) * jax.lax.rsqrt(deg_c)
    p = jnp.dot(s, s, preferred_element_type=jnp.float32, precision=_HI)
    r = jnp.dot(p, gmat_ref[...], preferred_element_type=jnp.float32, precision=_HI)
    t1 = jnp.dot(e_ref[...], r, preferred_element_type=jnp.float32, precision=_HI)
    w0 = jnp.dot(t1 * mask_ref[...], jrep_ref[...],
                 preferred_element_type=jnp.float32, precision=_HI)

    w_acc[...] = w0 * (gt_ref[...] * _INV_SQRT1P)
    c0_acc[...] = (jnp.dot(bt_ref[...], w0, preferred_element_type=jnp.float32,
                           precision=_HI)
                   + bias1_ref[...])


def _fused_kernel(a_ref, gmat_ref, e_ref, mask_ref, jrep_ref, gt_ref, bt_ref,
                  bias1_ref, x_ref, w2_ref, b2_ref, o_ref, w_acc, c0_acc):
    @pl.when(pl.program_id(0) == 0)
    def _():
        _fold(a_ref, gmat_ref, e_ref, mask_ref, jrep_ref, gt_ref, bt_ref,
              bias1_ref, w_acc, c0_acc)

    w = w_acc[...]
    y = jnp.dot(x_ref[...], w[:NF, :], preferred_element_type=jnp.float32)
    y = jnp.maximum(y + c0_acc[...], 0.0)
    o_ref[...] = (jnp.dot(y, w2_ref[...], preferred_element_type=jnp.float32)
                  + b2_ref[...])


def kernel(X, edge_weight, bn_gamma, bn_beta, Wc, bc, W1, b1, W2, b2):
    B = X.shape[0]
    X_flat = X.reshape(B, NF)
    a_dense = _sc_scatter_build(edge_weight).reshape(NP, NP)
    W1r = W1.reshape(O1, N, NH)
    G = jnp.einsum('onh,hf->nfo', W1r, Wc)
    Gmat = jnp.pad(G.reshape(N, F * O1), ((0, NP - N), (0, 0)))
    bias1 = (b1 + jnp.einsum('onh,h->o', W1r, bc)).reshape(1, O1)
    gt = jnp.pad(jnp.tile(bn_gamma, N), (0, NFP - NF)).reshape(NFP, 1)
    bt = jnp.pad(jnp.tile(bn_beta, N), (0, NFP - NF)).reshape(1, NFP)

    BT = 1024
    cblk = lambda i: (0, 0)
    out = pl.pallas_call(
        _fused_kernel,
        grid=(B // BT,),
        in_specs=[pl.BlockSpec((NP, NP), cblk),
                  pl.BlockSpec((NP, F * O1), cblk),
                  pl.BlockSpec((NFP, NP), cblk),
                  pl.BlockSpec((NFP, NFP), cblk),
                  pl.BlockSpec((NFP, O1), cblk),
                  pl.BlockSpec((NFP, 1), cblk),
                  pl.BlockSpec((1, NFP), cblk),
                  pl.BlockSpec((1, O1), cblk),
                  pl.BlockSpec((BT, NF), lambda i: (i, 0)),
                  pl.BlockSpec((O1, NC), cblk),
                  pl.BlockSpec((1, NC), cblk)],
        out_specs=pl.BlockSpec((BT, NC), lambda i: (i, 0)),
        out_shape=jax.ShapeDtypeStruct((B, NC), jnp.float32),
        scratch_shapes=[pltpu.VMEM((NFP, O1), jnp.float32),
                        pltpu.VMEM((1, O1), jnp.float32)],
    )(a_dense, Gmat, jnp.asarray(_E_SEL), jnp.asarray(_MASK), jnp.asarray(_JREP),
      gt, bt, bias1, X_flat, W2.T, b2.reshape(1, NC))
    return out


# speedup vs baseline: 1.2239x; 1.0301x over previous
import functools
import numpy as np
import jax
from jax import lax
import jax.numpy as jnp
from jax.experimental import pallas as pl
from jax.experimental.pallas import tpu as pltpu
from jax.experimental.pallas import tpu_sc as plsc

N = 62
F = 5
NH = 32
O1 = 64
NC = 3
NP = 64
NF = N * F
NFP = NP * F
NTRI = N * (N + 1) // 2
BN_EPS = 1e-5
_INV_SQRT1P = float(1.0 / np.sqrt(1.0 + BN_EPS))
_HI = jax.lax.Precision.HIGHEST

_r = np.arange(NFP)
_c = np.arange(NFP)
_E_SEL = np.zeros((NFP, NP), np.float32)
_E_SEL[_r, _r // F] = 1.0
_MASK = ((_r[:, None] % F) == (_c[None, :] // O1)).astype(np.float32)
_JREP = ((_c[:, None] % O1) == np.arange(O1)[None, :]).astype(np.float32)

@functools.partial(
    pl.kernel,
    mesh=plsc.VectorSubcoreMesh(core_axis_name="c", subcore_axis_name="s", num_cores=1),
    out_type=jax.ShapeDtypeStruct((NP * NP,), jnp.float32),
    scratch_types=[pltpu.VMEM((NTRI,), jnp.float32),
                   pltpu.VMEM((256,), jnp.float32)],
    compiler_params=pltpu.CompilerParams(needs_layout_passes=False),
)
def _sc_scatter_build(ew_hbm, out_hbm, ew_v, row_v):
    wid = lax.axis_index("s")
    pltpu.sync_copy(ew_hbm, ew_v)
    for rr in range(4):
        i = wid * 4 + rr
        tri_i = (i * (i + 1)) // 2
        for k in range(4):
            j = lax.iota(jnp.int32, 16) + (k * 16)
            low = tri_i + j
            high = ((j * (j + 1)) // 2) + i
            idx = jnp.where(j <= i, low, high)
            valid = (j < N) & (i < N)
            idx = jnp.where(valid, idx, 0)
            vals = plsc.load_gather(ew_v, [idx])
            vals = jnp.where(valid, jnp.maximum(vals, 0.0), 0.0)
            row_v[pl.ds(rr * 64 + k * 16, 16)] = vals
    pltpu.sync_copy(row_v, out_hbm.at[pl.ds(wid * 256, 256)])


def _fold(a_ref, gmat_ref, e_ref, mask_ref, jrep_ref, gt_ref, bt_ref,
          bias1_ref, w_acc, c0_acc):
    a = a_ref[...]
    eye = (jax.lax.broadcasted_iota(jnp.int32, (NP, NP), 0) ==
           jax.lax.broadcasted_iota(jnp.int32, (NP, NP), 1)).astype(jnp.float32)
    drow = jnp.sum(a, axis=1, keepdims=True)
    dcol = jnp.sum(a, axis=0, keepdims=True)
    l = a * jax.lax.rsqrt(drow + 1e-10) * jax.lax.rsqrt(dcol + 1e-10)
    la = jnp.abs(l)
    deg_r = jnp.sum(la, axis=1, keepdims=True) + 1.0
    deg_c = jnp.sum(la, axis=0, keepdims=True) + 1.0
    s = (l + eye) * jax.lax.rsqrt(deg_r) * jax.lax.rsqrt(deg_c)
    p = jnp.dot(s, s, preferred_element_type=jnp.float32, precision=_HI)
    r = jnp.dot(p, gmat_ref[...], preferred_element_type=jnp.float32, precision=_HI)
    t1 = jnp.dot(e_ref[...], r, preferred_element_type=jnp.float32)
    w0 = jnp.dot(t1 * mask_ref[...], jrep_ref[...],
                 preferred_element_type=jnp.float32)

    w_acc[...] = w0 * (gt_ref[...] * _INV_SQRT1P)
    c0_acc[...] = (jnp.dot(bt_ref[...], w0, preferred_element_type=jnp.float32,
                           precision=_HI)
                   + bias1_ref[...])


def _fused_kernel(a_ref, gmat_ref, e_ref, mask_ref, jrep_ref, gt_ref, bt_ref,
                  bias1_ref, x_ref, w2_ref, b2_ref, o_ref, w_acc, c0_acc):
    @pl.when(pl.program_id(0) == 0)
    def _():
        _fold(a_ref, gmat_ref, e_ref, mask_ref, jrep_ref, gt_ref, bt_ref,
              bias1_ref, w_acc, c0_acc)

    w = w_acc[...]
    y = jnp.dot(x_ref[...], w[:NF, :], preferred_element_type=jnp.float32)
    y = jnp.maximum(y + c0_acc[...], 0.0)
    o_ref[...] = (jnp.dot(y, w2_ref[...], preferred_element_type=jnp.float32)
                  + b2_ref[...])


def kernel(X, edge_weight, bn_gamma, bn_beta, Wc, bc, W1, b1, W2, b2):
    B = X.shape[0]
    X_flat = X.reshape(B, NF)
    a_dense = _sc_scatter_build(edge_weight).reshape(NP, NP)
    W1r = W1.reshape(O1, N, NH)
    G = jnp.einsum('onh,hf->nfo', W1r, Wc)
    Gmat = jnp.pad(G.reshape(N, F * O1), ((0, NP - N), (0, 0)))
    bias1 = (b1 + jnp.einsum('onh,h->o', W1r, bc)).reshape(1, O1)
    gt = jnp.pad(jnp.tile(bn_gamma, N), (0, NFP - NF)).reshape(NFP, 1)
    bt = jnp.pad(jnp.tile(bn_beta, N), (0, NFP - NF)).reshape(1, NFP)

    BT = 1024
    cblk = lambda i: (0, 0)
    out = pl.pallas_call(
        _fused_kernel,
        grid=(B // BT,),
        in_specs=[pl.BlockSpec((NP, NP), cblk),
                  pl.BlockSpec((NP, F * O1), cblk),
                  pl.BlockSpec((NFP, NP), cblk),
                  pl.BlockSpec((NFP, NFP), cblk),
                  pl.BlockSpec((NFP, O1), cblk),
                  pl.BlockSpec((NFP, 1), cblk),
                  pl.BlockSpec((1, NFP), cblk),
                  pl.BlockSpec((1, O1), cblk),
                  pl.BlockSpec((BT, NF), lambda i: (i, 0)),
                  pl.BlockSpec((O1, NC), cblk),
                  pl.BlockSpec((1, NC), cblk)],
        out_specs=pl.BlockSpec((BT, NC), lambda i: (i, 0)),
        out_shape=jax.ShapeDtypeStruct((B, NC), jnp.float32),
        scratch_shapes=[pltpu.VMEM((NFP, O1), jnp.float32),
                        pltpu.VMEM((1, O1), jnp.float32)],
    )(a_dense, Gmat, jnp.asarray(_E_SEL), jnp.asarray(_MASK), jnp.asarray(_JREP),
      gt, bt, bias1, X_flat, W2.T, b2.reshape(1, NC))
    return out


# speedup vs baseline: 1.2248x; 1.0007x over previous
import functools
import numpy as np
import jax
from jax import lax
import jax.numpy as jnp
from jax.experimental import pallas as pl
from jax.experimental.pallas import tpu as pltpu
from jax.experimental.pallas import tpu_sc as plsc

N = 62
F = 5
NH = 32
O1 = 64
NC = 3
NP = 64
NF = N * F
NFP = NP * F
NTRI = N * (N + 1) // 2
BN_EPS = 1e-5
_INV_SQRT1P = float(1.0 / np.sqrt(1.0 + BN_EPS))
_HI = jax.lax.Precision.HIGHEST

_r = np.arange(NFP)
_c = np.arange(NFP)
_E_SEL = np.zeros((NFP, NP), np.float32)
_E_SEL[_r, _r // F] = 1.0
_MASK = ((_r[:, None] % F) == (_c[None, :] // O1)).astype(np.float32)
_JREP = ((_c[:, None] % O1) == np.arange(O1)[None, :]).astype(np.float32)

@functools.lru_cache(maxsize=1)
def _sc_scatter_build():
    return functools.partial(
        pl.kernel,
        mesh=plsc.VectorSubcoreMesh(core_axis_name="c", subcore_axis_name="s",
                                    num_cores=1),
        out_type=jax.ShapeDtypeStruct((NP * NP,), jnp.float32),
        scratch_types=[pltpu.VMEM((NTRI,), jnp.float32),
                       pltpu.VMEM((256,), jnp.float32)],
        compiler_params=pltpu.CompilerParams(needs_layout_passes=False),
    )(_sc_scatter_body)


def _sc_scatter_body(ew_hbm, out_hbm, ew_v, row_v):
    wid = lax.axis_index("s")
    pltpu.sync_copy(ew_hbm, ew_v)
    for rr in range(4):
        i = wid * 4 + rr
        tri_i = (i * (i + 1)) // 2
        for k in range(4):
            j = lax.iota(jnp.int32, 16) + (k * 16)
            low = tri_i + j
            high = ((j * (j + 1)) // 2) + i
            idx = jnp.where(j <= i, low, high)
            valid = (j < N) & (i < N)
            idx = jnp.where(valid, idx, 0)
            vals = plsc.load_gather(ew_v, [idx])
            vals = jnp.where(valid, jnp.maximum(vals, 0.0), 0.0)
            row_v[pl.ds(rr * 64 + k * 16, 16)] = vals
    pltpu.sync_copy(row_v, out_hbm.at[pl.ds(wid * 256, 256)])


def _fold(a_ref, gmat_ref, e_ref, mask_ref, jrep_ref, gt_ref, bt_ref,
          bias1_ref, w_acc, c0_acc):
    a = a_ref[...]
    eye = (jax.lax.broadcasted_iota(jnp.int32, (NP, NP), 0) ==
           jax.lax.broadcasted_iota(jnp.int32, (NP, NP), 1)).astype(jnp.float32)
    drow = jnp.sum(a, axis=1, keepdims=True)
    dcol = jnp.sum(a, axis=0, keepdims=True)
    l = a * jax.lax.rsqrt(drow + 1e-10) * jax.lax.rsqrt(dcol + 1e-10)
    la = jnp.abs(l)
    deg_r = jnp.sum(la, axis=1, keepdims=True) + 1.0
    deg_c = jnp.sum(la, axis=0, keepdims=True) + 1.0
    s = (l + eye) * jax.lax.rsqrt(deg_r) * jax.lax.rsqrt(deg_c)
    p = jnp.dot(s, s, preferred_element_type=jnp.float32, precision=_HI)
    r = jnp.dot(p, gmat_ref[...], preferred_element_type=jnp.float32, precision=_HI)
    t1 = jnp.dot(e_ref[...], r, preferred_element_type=jnp.float32)
    w0 = jnp.dot(t1 * mask_ref[...], jrep_ref[...],
                 preferred_element_type=jnp.float32)

    w_acc[...] = w0 * (gt_ref[...] * _INV_SQRT1P)
    c0_acc[...] = (jnp.dot(bt_ref[...], w0, preferred_element_type=jnp.float32,
                           precision=_HI)
                   + bias1_ref[...])


def _fused_kernel(a_ref, gmat_ref, e_ref, mask_ref, jrep_ref, gt_ref, bt_ref,
                  bias1_ref, x_ref, w2_ref, b2_ref, o_ref, w_acc, c0_acc):
    @pl.when(pl.program_id(0) == 0)
    def _():
        _fold(a_ref, gmat_ref, e_ref, mask_ref, jrep_ref, gt_ref, bt_ref,
              bias1_ref, w_acc, c0_acc)

    w = w_acc[...]
    y = jnp.dot(x_ref[...], w[:NF, :], preferred_element_type=jnp.float32)
    y = jnp.maximum(y + c0_acc[...], 0.0)
    o_ref[...] = (jnp.dot(y, w2_ref[...], preferred_element_type=jnp.float32)
                  + b2_ref[...])


def kernel(X, edge_weight, bn_gamma, bn_beta, Wc, bc, W1, b1, W2, b2):
    B = X.shape[0]
    X_flat = X.reshape(B, NF)
    a_dense = _sc_scatter_build()(edge_weight).reshape(NP, NP)
    W1r = W1.reshape(O1, N, NH)
    G = jnp.einsum('onh,hf->nfo', W1r, Wc)
    Gmat = jnp.pad(G.reshape(N, F * O1), ((0, NP - N), (0, 0)))
    bias1 = (b1 + jnp.einsum('onh,h->o', W1r, bc)).reshape(1, O1)
    gt = jnp.pad(jnp.tile(bn_gamma, N), (0, NFP - NF)).reshape(NFP, 1)
    bt = jnp.pad(jnp.tile(bn_beta, N), (0, NFP - NF)).reshape(1, NFP)

    BT = 1024
    cblk = lambda i: (0, 0)
    out = pl.pallas_call(
        _fused_kernel,
        grid=(B // BT,),
        in_specs=[pl.BlockSpec((NP, NP), cblk),
                  pl.BlockSpec((NP, F * O1), cblk),
                  pl.BlockSpec((NFP, NP), cblk),
                  pl.BlockSpec((NFP, NFP), cblk),
                  pl.BlockSpec((NFP, O1), cblk),
                  pl.BlockSpec((NFP, 1), cblk),
                  pl.BlockSpec((1, NFP), cblk),
                  pl.BlockSpec((1, O1), cblk),
                  pl.BlockSpec((BT, NF), lambda i: (i, 0)),
                  pl.BlockSpec((O1, NC), cblk),
                  pl.BlockSpec((1, NC), cblk)],
        out_specs=pl.BlockSpec((BT, NC), lambda i: (i, 0)),
        out_shape=jax.ShapeDtypeStruct((B, NC), jnp.float32),
        scratch_shapes=[pltpu.VMEM((NFP, O1), jnp.float32),
                        pltpu.VMEM((1, O1), jnp.float32)],
    )(a_dense, Gmat, jnp.asarray(_E_SEL), jnp.asarray(_MASK), jnp.asarray(_JREP),
      gt, bt, bias1, X_flat, W2.T, b2.reshape(1, NC))
    return out
